# baseline (speedup 1.0000x reference)
import functools

import jax
import jax.numpy as jnp
import numpy as np
from jax import lax
from jax.experimental import pallas as pl
from jax.experimental.pallas import tpu as pltpu
from jax.experimental.pallas import tpu_sc as plsc

_B = 16384
_E = 32
_NNEG = 15
_REPS = _NNEG + 1
_OUT = _B * _REPS
_NW = 32
_EI = _B // 8


def _shift_table():
    rng = np.random.default_rng(0)
    picks = rng.choice(np.arange(1, _B), size=_NNEG, replace=False)
    return [0] + [int(a) for a in picks]


_SHIFTS = _shift_table()

_mesh = plsc.VectorSubcoreMesh(core_axis_name="c", subcore_axis_name="s")


@functools.partial(
    pl.kernel,
    out_type=jax.ShapeDtypeStruct((_E, _OUT), jnp.float32),
    mesh=_mesh,
    scratch_types=[
        pltpu.VMEM((_E, _EI), jnp.float32),
        pltpu.VMEM_SHARED((_E, 4 * _EI), jnp.float32),
    ]
    + [pltpu.SemaphoreType.DMA for _ in range(4)],
)
def _sc_tile_query(qt_hbm, qout_hbm, buf, shared, *sems):
    wid = lax.axis_index("s") * 2 + lax.axis_index("c")
    for w in range(8):
        e = w

        @pl.when(wid == w)
        def _(e=e):
            pltpu.sync_copy(qt_hbm.at[:, pl.ds(e * _EI, _EI)], buf)
            pltpu.sync_copy(buf, shared.at[:, pl.ds((e // 2) * _EI, _EI)])

    plsc.subcore_barrier()

    for w in range(8, _NW):
        e = w % 8

        @pl.when(wid == w)
        def _(e=e):
            pltpu.sync_copy(shared.at[:, pl.ds((e // 2) * _EI, _EI)], buf)

    for w in range(_NW):
        e, g = w % 8, w // 8

        @pl.when(wid == w)
        def _(e=e, g=g):
            cps = []
            for j in range(4):
                k = g * 4 + j
                cps.append(
                    pltpu.async_copy(
                        buf,
                        qout_hbm.at[:, pl.ds(k * _B + e * _EI, _EI)],
                        sems[j],
                    )
                )
            for c in cps:
                c.wait()


def _tc_body(item_ref, shifts_ref, out_ref, scratch_ref):
    k = pl.program_id(0)

    @pl.when(k == 0)
    def _():
        scratch_ref[:, pl.ds(0, _B)] = item_ref[...]
        scratch_ref[:, pl.ds(_B, _B)] = item_ref[...]

    for h in range(4):
        a = shifts_ref[4 * k + h]
        a_hi = pl.multiple_of((a // 128) * 128, 128)
        r = a - a_hi
        coarse = scratch_ref[:, pl.ds(a_hi, _B)]
        out_ref[:, pl.ds(h * _B, _B)] = pltpu.roll(coarse, (_B - r) % _B, 1)


def _tc_roll_items(item_t):
    shifts = jnp.asarray(_SHIFTS, jnp.int32)
    return pl.pallas_call(
        _tc_body,
        grid=(_REPS // 4,),
        in_specs=[
            pl.BlockSpec((_E, _B), lambda k: (0, 0)),
            pl.BlockSpec(memory_space=pltpu.SMEM),
        ],
        out_specs=pl.BlockSpec((_E, 4 * _B), lambda k: (0, k)),
        out_shape=jax.ShapeDtypeStruct((_E, _OUT), jnp.float32),
        scratch_shapes=[pltpu.VMEM((_E, 2 * _B), jnp.float32)],
    )(item_t, shifts)


def kernel(query_embeddings, item_embeddings):
    q_out_t = _sc_tile_query(query_embeddings.T)
    it_out_t = _tc_roll_items(item_embeddings.T)
    return q_out_t.T, it_out_t.T


# speedup vs baseline: 1.0269x; 1.0269x over previous
import functools

import jax
import jax.numpy as jnp
import numpy as np
from jax import lax
from jax.experimental import pallas as pl
from jax.experimental.pallas import tpu as pltpu
from jax.experimental.pallas import tpu_sc as plsc

_B = 16384
_E = 32
_NNEG = 15
_REPS = _NNEG + 1
_OUT = _B * _REPS
_NW = 32
_EI = _B // 8


def _shift_table():
    rng = np.random.default_rng(0)
    picks = rng.choice(np.arange(1, _B), size=_NNEG, replace=False)
    return [0] + [int(a) for a in picks]


_SHIFTS = _shift_table()

_mesh = plsc.VectorSubcoreMesh(core_axis_name="c", subcore_axis_name="s")


@functools.partial(
    pl.kernel,
    out_type=jax.ShapeDtypeStruct((_E, _OUT), jnp.float32),
    mesh=_mesh,
    scratch_types=[
        pltpu.VMEM((_E, _EI), jnp.float32),
        pltpu.VMEM_SHARED((_E, 4 * _EI), jnp.float32),
    ]
    + [pltpu.SemaphoreType.DMA for _ in range(4)],
)
def _sc_tile_query(qt_hbm, qout_hbm, buf, shared, *sems):
    wid = lax.axis_index("s") * 2 + lax.axis_index("c")
    for w in range(8):
        e = w

        @pl.when(wid == w)
        def _(e=e):
            pltpu.sync_copy(qt_hbm.at[:, pl.ds(e * _EI, _EI)], buf)
            pltpu.sync_copy(buf, shared.at[:, pl.ds((e // 2) * _EI, _EI)])

    plsc.subcore_barrier()

    for w in range(8, _NW):
        e = w % 8

        @pl.when(wid == w)
        def _(e=e):
            pltpu.sync_copy(shared.at[:, pl.ds((e // 2) * _EI, _EI)], buf)

    for w in range(_NW):
        e, g = w % 8, w // 8

        @pl.when(wid == w)
        def _(e=e, g=g):
            cps = []
            for j in range(4):
                k = g * 4 + j
                cps.append(
                    pltpu.async_copy(
                        buf,
                        qout_hbm.at[:, pl.ds(k * _B + e * _EI, _EI)],
                        sems[j],
                    )
                )
            for c in cps:
                c.wait()


def _tc_body(item_ref, shifts_ref, out_ref):
    k = pl.program_id(0)
    for h in range(2):
        a = shifts_ref[2 * k + h]
        out_ref[:, pl.ds(h * _B, _B)] = pltpu.roll(
            item_ref[...], lax.rem(_B - a, _B), 1
        )


def _tc_roll_items(item_t):
    shifts = jnp.asarray(_SHIFTS, jnp.int32)
    return pl.pallas_call(
        _tc_body,
        grid=(_REPS // 2,),
        in_specs=[
            pl.BlockSpec((_E, _B), lambda k: (0, 0)),
            pl.BlockSpec(memory_space=pltpu.SMEM),
        ],
        out_specs=pl.BlockSpec((_E, 2 * _B), lambda k: (0, k)),
        out_shape=jax.ShapeDtypeStruct((_E, _OUT), jnp.float32),
    )(item_t, shifts)


def kernel(query_embeddings, item_embeddings):
    q_out_t = _sc_tile_query(query_embeddings.T)
    it_out_t = _tc_roll_items(item_embeddings.T)
    return q_out_t.T, it_out_t.T
